# baseline (speedup 1.0000x reference)
import functools

import jax
import jax.numpy as jnp
from jax.experimental import pallas as pl
from jax.experimental.pallas import tpu as pltpu

E = 8
D_IN = 6
H = 256
D_OUT = 4
MARGIN = 1.25
T = 256


def _routing_weights(xt, c):
    d2 = jnp.zeros((xt.shape[0], E), dtype=jnp.float32)
    for j in range(3):
        diff = xt[:, j:j + 1] - c[:, j][None, :]
        d2 = d2 + diff * diff
    d = jnp.sqrt(d2)
    inv = 1.0 / (d + 1e-8)
    dmin = jnp.min(d, axis=1, keepdims=True)
    inv = jnp.where(d > MARGIN * dmin, 0.0, inv)
    return inv / jnp.sum(inv, axis=1, keepdims=True)


def _mlp_kernel(tile_byte_ref, x_ref, c_ref, w1_ref, b1_ref, w2_ref, b2_ref,
                w3_ref, b3_ref, out_ref):
    xt = x_ref[...]
    w = _routing_weights(xt, c_ref[...])
    tb = tile_byte_ref[pl.program_id(0)]
    out_ref[...] = jnp.zeros((xt.shape[0], D_OUT), jnp.float32)
    for e in range(E):
        @pl.when(((tb >> e) & 1) != 0)
        def _(e=e):
            h = jnp.dot(xt, w1_ref[e], preferred_element_type=jnp.float32)
            h = jax.nn.relu(h + b1_ref[e][None, :])
            h = jnp.dot(h, w2_ref[e], preferred_element_type=jnp.float32)
            h = jax.nn.relu(h + b2_ref[e][None, :])
            o = jnp.dot(h, w3_ref[e], preferred_element_type=jnp.float32)
            o = o + b3_ref[e][None, :]
            out_ref[...] += o * w[:, e:e + 1]


@jax.jit
def kernel(x, centroids, W1, b1, W2, b2, W3, b3):
    n = x.shape[0]
    n_tiles = n // T

    diff = x[:, None, :3] - centroids[None, :, :]
    d = jnp.sqrt(jnp.sum(diff * diff, axis=-1))
    dmin = jnp.min(d, axis=1, keepdims=True)
    mask = d <= MARGIN * dmin
    key = jnp.sum(mask.astype(jnp.int32) * (1 << jnp.arange(E)), axis=1)
    perm = jnp.argsort(key)
    inv_perm = jnp.argsort(perm)
    key_s = key[perm]
    tile_byte = jax.lax.reduce(
        key_s.reshape(n_tiles, T), jnp.int32(0), jax.lax.bitwise_or, (1,))

    xp = jnp.pad(x, ((0, 0), (0, 8 - D_IN)))
    x_s = xp[perm]
    W1p = jnp.pad(W1, ((0, 0), (0, 8 - D_IN), (0, 0)))

    grid_spec = pltpu.PrefetchScalarGridSpec(
        num_scalar_prefetch=1,
        grid=(n_tiles,),
        in_specs=[
            pl.BlockSpec((T, 8), lambda i, tb: (i, 0)),
            pl.BlockSpec((E, 3), lambda i, tb: (0, 0)),
            pl.BlockSpec((E, 8, H), lambda i, tb: (0, 0, 0)),
            pl.BlockSpec((E, H), lambda i, tb: (0, 0)),
            pl.BlockSpec((E, H, H), lambda i, tb: (0, 0, 0)),
            pl.BlockSpec((E, H), lambda i, tb: (0, 0)),
            pl.BlockSpec((E, H, D_OUT), lambda i, tb: (0, 0, 0)),
            pl.BlockSpec((E, D_OUT), lambda i, tb: (0, 0)),
        ],
        out_specs=pl.BlockSpec((T, D_OUT), lambda i, tb: (i, 0)),
    )
    out_s = pl.pallas_call(
        _mlp_kernel,
        grid_spec=grid_spec,
        out_shape=jax.ShapeDtypeStruct((n, D_OUT), jnp.float32),
    )(tile_byte, x_s, centroids, W1p, b1, W2, b2, W3, b3)
    return out_s[inv_perm]
